# baseline (speedup 1.0000x reference)
import jax
import jax.numpy as jnp
from jax.experimental import pallas as pl
from jax.experimental.pallas import tpu as pltpu

_NCHUNK = 8


def _copy_body(x_ref, o_ref, *sems):
    h = x_ref.shape[1]
    ch = h // _NCHUNK
    for i in range(_NCHUNK):
        pltpu.make_async_copy(
            x_ref.at[:, pl.ds(i * ch, ch)],
            o_ref.at[:, pl.ds(i * ch, ch)],
            sems[i],
        ).start()
    for i in range(_NCHUNK):
        pltpu.make_async_copy(
            x_ref.at[:, pl.ds(i * ch, ch)],
            o_ref.at[:, pl.ds(i * ch, ch)],
            sems[i],
        ).wait()


def kernel(x):
    n, c, h, w = x.shape
    y = jnp.transpose(x, (1, 2, 3, 0))
    out = pl.pallas_call(
        _copy_body,
        out_shape=jax.ShapeDtypeStruct((c, h, w, n), x.dtype),
        in_specs=[pl.BlockSpec(memory_space=pltpu.MemorySpace.HBM)],
        out_specs=pl.BlockSpec(memory_space=pltpu.MemorySpace.HBM),
        scratch_shapes=[pltpu.SemaphoreType.DMA] * _NCHUNK,
    )(y)
    return jnp.transpose(out, (3, 0, 1, 2))


# speedup vs baseline: 47.7274x; 47.7274x over previous
import jax
import jax.numpy as jnp
from jax.experimental import pallas as pl

_BH = 14


def _copy_body(x_ref, o_ref):
    o_ref[...] = x_ref[...]


def kernel(x):
    n, c, h, w = x.shape
    y = jnp.transpose(x, (1, 2, 3, 0))
    out = pl.pallas_call(
        _copy_body,
        out_shape=jax.ShapeDtypeStruct((c, h, w, n), x.dtype),
        grid=(h // _BH,),
        in_specs=[pl.BlockSpec((c, _BH, w, n), lambda i: (0, i, 0, 0))],
        out_specs=pl.BlockSpec((c, _BH, w, n), lambda i: (0, i, 0, 0)),
    )(y)
    return jnp.transpose(out, (3, 0, 1, 2))


# speedup vs baseline: 49.1526x; 1.0299x over previous
import jax
import jax.numpy as jnp
from jax.experimental import pallas as pl

_BH = 32


def _copy_body(x_ref, o_ref):
    o_ref[...] = x_ref[...]


def kernel(x):
    n, c, h, w = x.shape
    y = jnp.transpose(x, (1, 2, 3, 0))
    out = pl.pallas_call(
        _copy_body,
        out_shape=jax.ShapeDtypeStruct((c, h, w, n), x.dtype),
        grid=(h // _BH,),
        in_specs=[pl.BlockSpec((c, _BH, w, n), lambda i: (0, i, 0, 0))],
        out_specs=pl.BlockSpec((c, _BH, w, n), lambda i: (0, i, 0, 0)),
    )(y)
    return jnp.transpose(out, (3, 0, 1, 2))
